# baseline (speedup 1.0000x reference)
import functools

import jax
import jax.numpy as jnp
from jax import lax
from jax.experimental import pallas as pl
from jax.experimental.pallas import tpu as pltpu, tpu_sc as plsc

_NUM_CAMS = 100000
_D = 9
_CHUNK = 128
_FIRE = 12


@functools.cache
def _make_gather(B):
    info = plsc.get_sparse_core_info()
    NC, NS, L = info.num_cores, info.num_subcores, info.num_lanes
    NW = NC * NS
    b_per_w = B // NW
    e_per_w = b_per_w * _D
    n_chunks = e_per_w // _CHUNK
    n_steps = n_chunks // _FIRE
    assert B % NW == 0 and e_per_w % (_CHUNK * _FIRE) == 0
    mesh = plsc.VectorSubcoreMesh(core_axis_name="c", subcore_axis_name="s")

    @functools.partial(
        pl.kernel,
        mesh=mesh,
        compiler_params=pltpu.CompilerParams(
            use_tc_tiling_on_sc=False, needs_layout_passes=False
        ),
        out_type=jax.ShapeDtypeStruct((B * _D,), jnp.float32),
        scratch_types=[
            pltpu.VMEM((b_per_w,), jnp.int32),
            pltpu.VMEM((e_per_w,), jnp.int32),
            pltpu.VMEM((e_per_w,), jnp.float32),
            pltpu.SemaphoreType.DMA,
        ],
    )
    def k(idx_hbm, table_hbm, out_hbm, idx_v, idx9_v, rows_v, sem):
        wid = lax.axis_index("s") * NC + lax.axis_index("c")
        pltpu.sync_copy(idx_hbm.at[pl.ds(wid * b_per_w, b_per_w)], idx_v)
        lane9 = lax.iota(jnp.int32, L) * _D
        for v in range(b_per_w // L):
            c9 = idx_v[pl.ds(v * L, L)] * _D
            pb = lane9 + (v * L * _D)
            for j in range(_D):
                plsc.store_scatter(idx9_v, [pb + j], c9 + j)

        def body(step, carry):
            copies = [
                pltpu.async_copy(
                    table_hbm.at[idx9_v.at[pl.ds((step * _FIRE + b) * _CHUNK, _CHUNK)]],
                    rows_v.at[pl.ds((step * _FIRE + b) * _CHUNK, _CHUNK)],
                    sem,
                )
                for b in range(_FIRE)
            ]
            for c in copies:
                c.wait()
            return carry

        lax.fori_loop(0, n_steps, body, 0)
        pltpu.sync_copy(rows_v, out_hbm.at[pl.ds(wid * e_per_w, e_per_w)])

    return k


def kernel(i, param):
    B = i.shape[0]
    table = param.reshape(-1)
    out = _make_gather(B)(i.astype(jnp.int32), table)
    return out.reshape(B, 3, 3)


# speedup vs baseline: 9.9773x; 9.9773x over previous
import functools

import jax
import jax.numpy as jnp
from jax import lax
from jax.experimental import pallas as pl
from jax.experimental.pallas import tpu as pltpu, tpu_sc as plsc

_NUM_CAMS = 100000
_D = 9
_CHUNK = 128
_FIRE = 12


@functools.cache
def _make_gather(B):
    info = plsc.get_sparse_core_info()
    NC, NS, L = info.num_cores, info.num_subcores, info.num_lanes
    NW = NC * NS
    b_per_w = B // NW
    e_per_w = b_per_w * _D
    n_chunks = e_per_w // _CHUNK
    n_steps = n_chunks // _FIRE
    assert B % NW == 0 and e_per_w % (_CHUNK * _FIRE) == 0
    mesh = plsc.VectorSubcoreMesh(core_axis_name="c", subcore_axis_name="s")

    @functools.partial(
        pl.kernel,
        mesh=mesh,
        compiler_params=pltpu.CompilerParams(
            use_tc_tiling_on_sc=False, needs_layout_passes=False
        ),
        out_type=jax.ShapeDtypeStruct((B * _D,), jnp.float32),
        scratch_types=[
            pltpu.VMEM((b_per_w,), jnp.int32),
            pltpu.VMEM((e_per_w,), jnp.int32),
            pltpu.VMEM((e_per_w,), jnp.float32),
            pltpu.SemaphoreType.DMA,
        ],
    )
    def k(idx_hbm, table_hbm, out_hbm, idx_v, idx9_v, rows_v, sem):
        wid = lax.axis_index("s") * NC + lax.axis_index("c")
        pltpu.sync_copy(idx_hbm.at[pl.ds(wid * b_per_w, b_per_w)], idx_v)
        for v in range(b_per_w // L):
            c = idx_v[pl.ds(v * L, L)]
            for kk in range(_D):
                idx9_v[pl.ds(kk * b_per_w + v * L, L)] = c + kk * _NUM_CAMS

        def body(step, carry):
            copies = [
                pltpu.async_copy(
                    table_hbm.at[idx9_v.at[pl.ds((step * _FIRE + b) * _CHUNK, _CHUNK)]],
                    rows_v.at[pl.ds((step * _FIRE + b) * _CHUNK, _CHUNK)],
                    sem,
                )
                for b in range(_FIRE)
            ]
            for c in copies:
                c.wait()
            return carry

        lax.fori_loop(0, n_steps, body, 0)
        for kk in range(_D):
            pltpu.sync_copy(
                rows_v.at[pl.ds(kk * b_per_w, b_per_w)],
                out_hbm.at[pl.ds(kk * B + wid * b_per_w, b_per_w)],
            )

    return k


def kernel(i, param):
    B = i.shape[0]
    table = param.transpose(1, 2, 0).reshape(-1)
    out = _make_gather(B)(i.astype(jnp.int32), table)
    return out.reshape(3, 3, B).transpose(2, 0, 1)
